# baseline (speedup 1.0000x reference)
import jax
import jax.numpy as jnp
from jax.experimental import pallas as pl
from jax.experimental.pallas import tpu as pltpu

_DU = 20
_DZ = 6
_DZU = _DU * _DZ
_BT = 128
_HI = jax.lax.Precision.HIGHEST


def _block_diag(blocks):
    g, k, n = blocks.shape
    eye = jnp.eye(g, dtype=blocks.dtype)
    bd = blocks[:, :, None, :] * eye[:, None, :, None]
    return bd.reshape(g * k, g * n)


def _expand_mid(W, b):
    wt = W.T[None].astype(jnp.float32)
    blocks = jnp.broadcast_to(wt, (_DU,) + wt.shape[1:])
    return _block_diag(blocks), jnp.tile(b, (_DU,))[None, :]


def _placed_rows(vals_w, vals_b, offset_per_group, width):
    g, k, w = vals_w.shape
    pad_w = jnp.pad(vals_w, ((0, 0), (0, 0), (0, _DZU - w)))
    pad_b = jnp.pad(vals_b, ((0, 0), (0, _DZU - w)))
    rows_w = jnp.stack(
        [jnp.roll(pad_w[i], offset_per_group[i], axis=-1) for i in range(g)])
    rows_b = jnp.stack(
        [jnp.roll(pad_b[i], offset_per_group[i], axis=-1) for i in range(g)])
    return rows_w, rows_b


def _build_weights(params1, params2):
    (W1a, b1a), (W2a, b2a), (W3a, b3a), (W4a, b4a) = params1
    (W1b, b1b), (W2b, b2b), (W3b, b3b), (W4b, b4b) = params2

    def first(W1, b1):
        blocks = jnp.broadcast_to(W1.T[None], (_DU, 3, 16))
        eye = jnp.eye(_DU, dtype=jnp.float32)
        bd = blocks[:, :, None, :] * eye[:, None, :, None]
        w = bd.transpose(1, 0, 2, 3).reshape(3 * _DU, 16 * _DU)
        return w, jnp.tile(b1, (_DU,))[None, :]

    W1Pa, B1a = first(W1a, b1a)
    W1Pb, B1b = first(W1b, b1b)
    W2Pa, B2a = _expand_mid(W2a, b2a)
    W2Pb, B2b = _expand_mid(W2b, b2b)
    W3Pa, B3a = _expand_mid(W3a, b3a)
    W3Pb, B3b = _expand_mid(W3b, b3b)

    f1_blocks = jnp.broadcast_to(W4a[0][None, :, None], (_DU, 16, 1))
    Wf1 = _block_diag(f1_blocks)
    Bf1 = jnp.tile(b4a[0][None], (_DU,))[None, :]

    f2_blocks = jnp.broadcast_to(W4b[:_DZ].T[None], (_DU, 16, _DZ))
    Wf2 = _block_diag(f2_blocks)
    Bf2 = jnp.tile(b4b[:_DZ], (_DU,))[None, :]

    g1_vals = jnp.broadcast_to(W4a[1:].T[None], (_DU, 16, 3 * _DZ))
    g1_bias = jnp.broadcast_to(b4a[1:][None], (_DU, 3 * _DZ))
    offs1 = [((i - 1) * _DZ) % _DZU for i in range(_DU)]
    rows_w, rows_b = _placed_rows(g1_vals, g1_bias, offs1, 3 * _DZ)
    Wg1 = _block_diag(rows_w)
    Bg1 = rows_b.reshape(1, _DU * _DZU)

    g2w = W4b[_DZ:].T.reshape(16, _DZ, 5 * _DZ)
    g2b = b4b[_DZ:].reshape(_DZ, 5 * _DZ)
    offs2 = [((jb - 2) * _DZ) % _DZU for jb in range(_DU)]
    blocks_w = []
    blocks_b = []
    for jb in range(_DU):
        pw = jnp.pad(g2w, ((0, 0), (0, 0), (0, _DZU - 5 * _DZ)))
        pb = jnp.pad(g2b, ((0, 0), (0, _DZU - 5 * _DZ)))
        rw = jnp.roll(pw, offs2[jb], axis=-1).reshape(16, _DZ * _DZU)
        rb = jnp.roll(pb, offs2[jb], axis=-1).reshape(_DZ * _DZU)
        blocks_w.append(rw)
        blocks_b.append(rb)
    Wg2 = _block_diag(jnp.stack(blocks_w))
    Bg2 = jnp.concatenate(blocks_b)[None, :]

    return (W1Pa, B1a, W2Pa, B2a, W3Pa, B3a, Wf1, Bf1, Wg1, Bg1,
            W1Pb, B1b, W2Pb, B2b, W3Pb, B3b, Wf2, Bf2, Wg2, Bg2)


def _body(x_ref,
          w1a, c1a, w2a, c2a, w3a, c3a, wf1, cf1, wg1, cg1,
          w1b, c1b, w2b, c2b, w3b, c3b, wf2, cf2, wg2, cg2,
          f1_ref, g1_ref, f2_ref, g2_ref):
    x = x_ref[...]
    xm = jnp.concatenate([x[:, -1:], x[:, :-1]], axis=1)
    xp = jnp.concatenate([x[:, 1:], x[:, :1]], axis=1)
    x3 = jnp.concatenate([xm, x, xp], axis=1)

    def mm(a, w, c):
        return jnp.dot(a, w[...], precision=_HI,
                       preferred_element_type=jnp.float32) + c[...]

    ha = jnp.maximum(mm(x3, w1a, c1a), 0.0)
    ha = jnp.maximum(mm(ha, w2a, c2a), 0.0)
    ha = jnp.maximum(mm(ha, w3a, c3a), 0.0)
    f1_ref[...] = mm(ha, wf1, cf1)
    g1_ref[...] = mm(ha, wg1, cg1)

    hb = jnp.maximum(mm(x3, w1b, c1b), 0.0)
    hb = jnp.maximum(mm(hb, w2b, c2b), 0.0)
    hb = jnp.maximum(mm(hb, w3b, c3b), 0.0)
    f2_ref[...] = mm(hb, wf2, cf2)
    g2_ref[...] = mm(hb, wg2, cg2)


def kernel(x, params1, params2):
    B = x.shape[0]
    bt = _BT if B % _BT == 0 else B
    grid = (B // bt,)

    wargs = _build_weights(params1, params2)

    x_spec = pl.BlockSpec((bt, _DU), lambda i: (i, 0))
    w_specs = [pl.BlockSpec(w.shape, lambda i: (0, 0)) for w in wargs]
    out_specs = [
        pl.BlockSpec((bt, _DU), lambda i: (i, 0)),
        pl.BlockSpec((bt, _DU * _DZU), lambda i: (i, 0)),
        pl.BlockSpec((bt, _DZU), lambda i: (i, 0)),
        pl.BlockSpec((bt, _DZU * _DZU), lambda i: (i, 0)),
    ]
    out_shape = [
        jax.ShapeDtypeStruct((B, _DU), jnp.float32),
        jax.ShapeDtypeStruct((B, _DU * _DZU), jnp.float32),
        jax.ShapeDtypeStruct((B, _DZU), jnp.float32),
        jax.ShapeDtypeStruct((B, _DZU * _DZU), jnp.float32),
    ]

    f1, g1, f2, g2 = pl.pallas_call(
        _body,
        grid=grid,
        in_specs=[x_spec] + list(w_specs),
        out_specs=out_specs,
        out_shape=out_shape,
        compiler_params=pltpu.CompilerParams(
            dimension_semantics=("parallel",),
            vmem_limit_bytes=100 * 1024 * 1024,
        ),
    )(x, *wargs)

    return (f1[..., None], g1.reshape(B, _DU, _DZU),
            f2[..., None], g2.reshape(B, _DZU, _DZU))


# speedup vs baseline: 1.3698x; 1.3698x over previous
import jax
import jax.numpy as jnp
from jax.experimental import pallas as pl
from jax.experimental.pallas import tpu as pltpu

_DU = 20
_DZ = 6
_DZU = _DU * _DZ
_BT = 128
_HI = jax.lax.Precision.HIGHEST
_MED = jax.lax.Precision.HIGHEST


def _block_diag(blocks):
    g, k, n = blocks.shape
    eye = jnp.eye(g, dtype=blocks.dtype)
    bd = blocks[:, :, None, :] * eye[:, None, :, None]
    return bd.reshape(g * k, g * n)


def _expand_mid(W, b):
    wt = W.T[None].astype(jnp.float32)
    blocks = jnp.broadcast_to(wt, (_DU,) + wt.shape[1:])
    return _block_diag(blocks), jnp.tile(b, (_DU,))[None, :]


def _build_weights(params1, params2):
    (W1a, b1a), (W2a, b2a), (W3a, b3a), (W4a, b4a) = params1
    (W1b, b1b), (W2b, b2b), (W3b, b3b), (W4b, b4b) = params2

    def first(W1, b1):
        blocks = jnp.broadcast_to(W1.T[None], (_DU, 3, 16))
        eye = jnp.eye(_DU, dtype=jnp.float32)
        bd = blocks[:, :, None, :] * eye[:, None, :, None]
        w = bd.transpose(1, 0, 2, 3).reshape(3 * _DU, 16 * _DU)
        return w, jnp.tile(b1, (_DU,))[None, :]

    W1Pa, B1a = first(W1a, b1a)
    W1Pb, B1b = first(W1b, b1b)
    W2Pa, B2a = _expand_mid(W2a, b2a)
    W2Pb, B2b = _expand_mid(W2b, b2b)
    W3Pa, B3a = _expand_mid(W3a, b3a)
    W3Pb, B3b = _expand_mid(W3b, b3b)

    f1_blocks = jnp.broadcast_to(W4a[0][None, :, None], (_DU, 16, 1))
    Wf1 = _block_diag(f1_blocks)
    Bf1 = jnp.tile(b4a[0][None], (_DU,))[None, :]

    f2_blocks = jnp.broadcast_to(W4b[:_DZ].T[None], (_DU, 16, _DZ))
    Wf2 = _block_diag(f2_blocks)
    Bf2 = jnp.tile(b4b[:_DZ], (_DU,))[None, :]

    g1w = jnp.pad(W4a[1:].T, ((0, 0), (0, _DZU - 3 * _DZ)))
    g1b = jnp.pad(b4a[1:], (0, _DZU - 3 * _DZ))
    Wg1 = jnp.stack([jnp.roll(g1w, ((i - 1) * _DZ) % _DZU, axis=-1)
                     for i in range(_DU)])
    Bg1 = jnp.stack([jnp.roll(g1b, ((i - 1) * _DZ) % _DZU)[None, :]
                     for i in range(_DU)])

    g2w = jnp.pad(W4b[_DZ:].T.reshape(16, _DZ, 5 * _DZ),
                  ((0, 0), (0, 0), (0, _DZU - 5 * _DZ)))
    g2b = jnp.pad(b4b[_DZ:].reshape(_DZ, 5 * _DZ),
                  ((0, 0), (0, _DZU - 5 * _DZ)))
    Wg2 = jnp.stack(
        [jnp.roll(g2w, ((jb - 2) * _DZ) % _DZU, axis=-1).reshape(16, 6 * _DZU)
         for jb in range(_DU)])
    Bg2 = jnp.stack(
        [jnp.roll(g2b, ((jb - 2) * _DZ) % _DZU, axis=-1).reshape(1, 6 * _DZU)
         for jb in range(_DU)])

    return (W1Pa, B1a, W2Pa, B2a, W3Pa, B3a, Wf1, Bf1, Wg1, Bg1,
            W1Pb, B1b, W2Pb, B2b, W3Pb, B3b, Wf2, Bf2, Wg2, Bg2)


def _body(x_ref,
          w1a, c1a, w2a, c2a, w3a, c3a, wf1, cf1, wg1, cg1,
          w1b, c1b, w2b, c2b, w3b, c3b, wf2, cf2, wg2, cg2,
          f1_ref, g1_ref, f2_ref, g2_ref):
    x = x_ref[...]
    xm = jnp.concatenate([x[:, -1:], x[:, :-1]], axis=1)
    xp = jnp.concatenate([x[:, 1:], x[:, :1]], axis=1)
    x3 = jnp.concatenate([xm, x, xp], axis=1)

    def mm(a, w, c, prec):
        return jnp.dot(a, w, precision=prec,
                       preferred_element_type=jnp.float32) + c

    ha = jnp.maximum(mm(x3, w1a[...], c1a[...], _HI), 0.0)
    ha = jnp.maximum(mm(ha, w2a[...], c2a[...], _HI), 0.0)
    ha = jnp.maximum(mm(ha, w3a[...], c3a[...], _HI), 0.0)
    f1_ref[...] = mm(ha, wf1[...], cf1[...], _HI)
    for i in range(_DU):
        g1_ref[:, _DZU * i:_DZU * (i + 1)] = mm(
            ha[:, 16 * i:16 * (i + 1)], wg1[i], cg1[i], _MED)

    hb = jnp.maximum(mm(x3, w1b[...], c1b[...], _HI), 0.0)
    hb = jnp.maximum(mm(hb, w2b[...], c2b[...], _HI), 0.0)
    hb = jnp.maximum(mm(hb, w3b[...], c3b[...], _HI), 0.0)
    f2_ref[...] = mm(hb, wf2[...], cf2[...], _HI)
    for jb in range(_DU):
        g2_ref[:, 6 * _DZU * jb:6 * _DZU * (jb + 1)] = mm(
            hb[:, 16 * jb:16 * (jb + 1)], wg2[jb], cg2[jb], _MED)


def kernel(x, params1, params2):
    B = x.shape[0]
    bt = _BT if B % _BT == 0 else B
    grid = (B // bt,)

    wargs = _build_weights(params1, params2)

    x_spec = pl.BlockSpec((bt, _DU), lambda i: (i, 0))
    w_specs = [pl.BlockSpec(w.shape, (lambda i: (0, 0)) if w.ndim == 2
               else (lambda i: (0, 0, 0))) for w in wargs]
    out_specs = [
        pl.BlockSpec((bt, _DU), lambda i: (i, 0)),
        pl.BlockSpec((bt, _DU * _DZU), lambda i: (i, 0)),
        pl.BlockSpec((bt, _DZU), lambda i: (i, 0)),
        pl.BlockSpec((bt, _DZU * _DZU), lambda i: (i, 0)),
    ]
    out_shape = [
        jax.ShapeDtypeStruct((B, _DU), jnp.float32),
        jax.ShapeDtypeStruct((B, _DU * _DZU), jnp.float32),
        jax.ShapeDtypeStruct((B, _DZU), jnp.float32),
        jax.ShapeDtypeStruct((B, _DZU * _DZU), jnp.float32),
    ]

    f1, g1, f2, g2 = pl.pallas_call(
        _body,
        grid=grid,
        in_specs=[x_spec] + list(w_specs),
        out_specs=out_specs,
        out_shape=out_shape,
        compiler_params=pltpu.CompilerParams(
            dimension_semantics=("parallel",),
            vmem_limit_bytes=100 * 1024 * 1024,
        ),
    )(x, *wargs)

    return (f1[..., None], g1.reshape(B, _DU, _DZU),
            f2[..., None], g2.reshape(B, _DZU, _DZU))
